# baseline (speedup 1.0000x reference)
import functools

import jax
import jax.numpy as jnp
from jax import lax
from jax.experimental import pallas as pl
from jax.experimental.pallas import tpu as pltpu
from jax.experimental.pallas import tpu_sc as plsc

B, T, D = 32, 2048, 256
K = 16
BSC = B - K
NW = 32
ROWS_PER_W = BSC * T // NW
CHUNK = 32
NCHUNK = ROWS_PER_W // CHUNK
DEPTH = 4
MAIN = (NCHUNK // DEPTH) * DEPTH
LANES = 16
NVREG = D // LANES


def _make_sc_add():
    mesh = plsc.VectorSubcoreMesh(core_axis_name="c", subcore_axis_name="s")

    @functools.partial(
        pl.kernel,
        mesh=mesh,
        out_type=jax.ShapeDtypeStruct((BSC * T, D), jnp.float32),
        scratch_types=[
            pltpu.VMEM((D,), jnp.float32),
            pltpu.VMEM((DEPTH, CHUNK, D), jnp.float32),
            pltpu.VMEM((DEPTH, CHUNK, D), jnp.float32),
        ] + [pltpu.SemaphoreType.DMA] * (2 * DEPTH),
    )
    def sc_add(sp_hbm, x_hbm, out_hbm, sp_v, ibuf, obuf, *sems):
        isems = sems[:DEPTH]
        osems = sems[DEPTH:]
        cid = lax.axis_index("c")
        sid = lax.axis_index("s")
        wid = sid * 2 + cid
        base = wid * ROWS_PER_W
        xbase = K * T + base

        pltpu.sync_copy(sp_hbm.at[K + lax.div(base, T)], sp_v)
        sp = [sp_v[pl.ds(LANES * g, LANES)] for g in range(NVREG)]

        def in_copy(c, s):
            return pltpu.make_async_copy(
                x_hbm.at[pl.ds(xbase + c * CHUNK, CHUNK)], ibuf.at[s], isems[s])

        def out_copy(c, s):
            return pltpu.make_async_copy(
                obuf.at[s], out_hbm.at[pl.ds(base + c * CHUNK, CHUNK)], osems[s])

        for s in range(DEPTH):
            in_copy(s, s).start()

        def compute(s):
            def row_body(r, carry):
                for g in range(NVREG):
                    obuf[s, r, pl.ds(LANES * g, LANES)] = (
                        ibuf[s, r, pl.ds(LANES * g, LANES)] + sp[g])
                return carry
            lax.fori_loop(0, CHUNK, row_body, 0, unroll=2)

        def step(c, s):
            in_copy(c, s).wait()

            @pl.when(c >= DEPTH)
            def _():
                out_copy(c, s).wait()

            compute(s)
            out_copy(c, s).start()

            @pl.when(c + DEPTH < NCHUNK)
            def _():
                in_copy(c + DEPTH, s).start()

        def super_body(i, carry):
            c0 = DEPTH * i
            for s in range(DEPTH):
                step(c0 + s, s)
            return carry

        lax.fori_loop(0, MAIN // DEPTH, super_body, 0)
        for c in range(MAIN, NCHUNK):
            s = c % DEPTH
            in_copy(c, s).wait()
            out_copy(c, s).wait()
            compute(s)
            out_copy(c, s).start()
        for c in range(NCHUNK - DEPTH, NCHUNK):
            out_copy(c, c % DEPTH).wait()

    return sc_add


_sc_add = _make_sc_add()

_TBLK = 512


def _tc_body(sp_ref, x_ref, o_ref):
    o_ref[...] = x_ref[...] + sp_ref[...]


_tc_add = pl.pallas_call(
    _tc_body,
    grid=(K, T // _TBLK),
    in_specs=[
        pl.BlockSpec((1, 1, D), lambda b, t: (b, 0, 0)),
        pl.BlockSpec((1, _TBLK, D), lambda b, t: (b, t, 0)),
    ],
    out_specs=pl.BlockSpec((1, _TBLK, D), lambda b, t: (b, t, 0)),
    out_shape=jax.ShapeDtypeStruct((K, T, D), jnp.float32),
)


def kernel(spembs, x):
    sc_out = _sc_add(spembs, x.reshape(B * T, D))
    tc_out = _tc_add(spembs.reshape(B, 1, D), x)
    out = jnp.concatenate([tc_out.reshape(K * T, D), sc_out], axis=0)
    return out.reshape(B, T, D)


# speedup vs baseline: 1.8414x; 1.8414x over previous
import functools

import jax
import jax.numpy as jnp
from jax import lax
from jax.experimental import pallas as pl
from jax.experimental.pallas import tpu as pltpu
from jax.experimental.pallas import tpu_sc as plsc

B, T, D = 32, 2048, 256
NW = 32
ROWS_PER_W = B * T // NW
CHUNK = 32
NCHUNK = ROWS_PER_W // CHUNK
DEPTH = 4
MAIN = (NCHUNK // DEPTH) * DEPTH
LANES = 16
NVREG = D // LANES


def _make_sc_add():
    mesh = plsc.VectorSubcoreMesh(core_axis_name="c", subcore_axis_name="s")

    @functools.partial(
        pl.kernel,
        mesh=mesh,
        out_type=jax.ShapeDtypeStruct((B * T, D), jnp.float32),
        scratch_types=[
            pltpu.VMEM((D,), jnp.float32),
            pltpu.VMEM((DEPTH, CHUNK, D), jnp.float32),
            pltpu.VMEM((DEPTH, CHUNK, D), jnp.float32),
        ] + [pltpu.SemaphoreType.DMA] * (2 * DEPTH),
    )
    def sc_add(sp_hbm, x_hbm, out_hbm, sp_v, ibuf, obuf, *sems):
        isems = sems[:DEPTH]
        osems = sems[DEPTH:]
        cid = lax.axis_index("c")
        sid = lax.axis_index("s")
        wid = sid * 2 + cid
        base = wid * ROWS_PER_W

        pltpu.sync_copy(sp_hbm.at[wid], sp_v)
        sp = [sp_v[pl.ds(LANES * g, LANES)] for g in range(NVREG)]

        def in_copy(c, s):
            return pltpu.make_async_copy(
                x_hbm.at[pl.ds(base + c * CHUNK, CHUNK)], ibuf.at[s], isems[s])

        def out_copy(c, s):
            return pltpu.make_async_copy(
                obuf.at[s], out_hbm.at[pl.ds(base + c * CHUNK, CHUNK)], osems[s])

        for s in range(DEPTH):
            in_copy(s, s).start()

        def compute(s):
            def row_body(r, carry):
                for g in range(NVREG):
                    obuf[s, r, pl.ds(LANES * g, LANES)] = (
                        ibuf[s, r, pl.ds(LANES * g, LANES)] + sp[g])
                return carry
            lax.fori_loop(0, CHUNK, row_body, 0, unroll=2)

        def step(c, s):
            in_copy(c, s).wait()

            @pl.when(c >= DEPTH)
            def _():
                out_copy(c, s).wait()

            compute(s)
            out_copy(c, s).start()

            @pl.when(c + DEPTH < NCHUNK)
            def _():
                in_copy(c + DEPTH, s).start()

        def super_body(i, carry):
            c0 = DEPTH * i
            for s in range(DEPTH):
                step(c0 + s, s)
            return carry

        lax.fori_loop(0, MAIN // DEPTH, super_body, 0)
        for c in range(MAIN, NCHUNK):
            s = c % DEPTH
            in_copy(c, s).wait()
            out_copy(c, s).wait()
            compute(s)
            out_copy(c, s).start()
        for c in range(NCHUNK - DEPTH, NCHUNK):
            out_copy(c, c % DEPTH).wait()

    return sc_add


_sc_add = _make_sc_add()


def kernel(spembs, x):
    out = _sc_add(spembs, x.reshape(B * T, D))
    return out.reshape(B, T, D)
